# baseline (speedup 1.0000x reference)
import jax
import jax.numpy as jnp
from jax import lax
from jax.experimental import pallas as pl
from jax.experimental.pallas import tpu as pltpu
from jax.experimental.pallas import tpu_sc as plsc

_NUM_CLASSES = 1000000
_HIDDEN = 64
_BATCH = 16384

_NC = 2
_NS = 16
_LANES = 16
_NW = _NC * _NS
_BPW = _BATCH // _NW
_CHUNK = 128
_NCHUNK = _BPW // _CHUNK


def _emb_kernel(labels_hbm, drops_hbm, tpair_hbm, dr_hbm, out_hbm,
                lab_v, drops_v, pidx_v, dr_v, prow_v, outT_v, gsem, dsem):
    wid = lax.axis_index("s") * _NC + lax.axis_index("c")
    base = wid * _BPW
    iota = lax.iota(jnp.int32, _LANES)
    zeros = jnp.zeros((_LANES,), jnp.int32)

    pltpu.sync_copy(labels_hbm.at[wid], lab_v)
    pltpu.sync_copy(drops_hbm.at[wid], drops_v)
    pltpu.sync_copy(dr_hbm, dr_v)

    def mkidx(g, carry):
        l = lab_v[g // 8, pl.ds((g % 8) * _LANES, _LANES)]
        pidx_v[g // 8, pl.ds((g % 8) * _LANES, _LANES)] = l >> 1
        return carry

    lax.fori_loop(0, _BPW // _LANES, mkidx, 0)

    def fire(j, slot):
        return pltpu.async_copy(tpair_hbm.at[pidx_v.at[j]],
                                prow_v.at[slot], gsem)

    cp = fire(0, 0)
    for j in range(_NCHUNK):
        nxt = fire(j + 1, (j + 1) % 2) if j + 1 < _NCHUNK else None
        cp.wait()
        slot = j % 2
        prow2 = prow_v.at[slot]
        outT = outT_v.at[slot]

        def asm(g, carry):
            off = g * _LANES
            lvec = lab_v[j, pl.ds(off, _LANES)]
            dvec = drops_v[j, pl.ds(off, _LANES)]
            hv = (lvec & 1) * _HIDDEN
            rowidx = off + iota
            dmask = dvec == 1

            def percol(c, carry2):
                val = plsc.load_gather(prow2, [rowidx, hv + c])
                drc = plsc.load_gather(dr_v, [zeros, zeros + c])
                outT[c, pl.ds(off, _LANES)] = jnp.where(dmask, drc, val)
                return carry2

            lax.fori_loop(0, _HIDDEN, percol, 0)
            return carry

        lax.fori_loop(0, _CHUNK // _LANES, asm, 0)
        pltpu.sync_copy(outT, out_hbm.at[:, pl.ds(base + j * _CHUNK, _CHUNK)])
        cp = nxt


@jax.jit
def _embed(labels, drops, tpair, dr):
    mesh = plsc.VectorSubcoreMesh(core_axis_name="c", subcore_axis_name="s")
    return pl.kernel(
        _emb_kernel,
        mesh=mesh,
        out_type=jax.ShapeDtypeStruct((_HIDDEN, _BATCH), jnp.float32),
        scratch_types=[
            pltpu.VMEM((_NCHUNK, _CHUNK), jnp.int32),
            pltpu.VMEM((_NCHUNK, _CHUNK), jnp.int32),
            pltpu.VMEM((_NCHUNK, _CHUNK), jnp.int32),
            pltpu.VMEM((1, _HIDDEN), jnp.float32),
            pltpu.VMEM((2, _CHUNK, 2 * _HIDDEN), jnp.float32),
            pltpu.VMEM((2, _HIDDEN, _CHUNK), jnp.float32),
            pltpu.SemaphoreType.DMA,
            pltpu.SemaphoreType.DMA,
        ],
        compiler_params=pltpu.CompilerParams(needs_layout_passes=False,
                                             skip_device_barrier=True),
    )(labels, drops, tpair, dr)


def kernel(class_labels, train, force_drop_ids, table):
    del train
    labels3 = class_labels.astype(jnp.int32).reshape(_NW, _NCHUNK, _CHUNK)
    drops3 = force_drop_ids.astype(jnp.int32).reshape(_NW, _NCHUNK, _CHUNK)
    dr = table[_NUM_CLASSES].reshape(1, _HIDDEN)
    scale = (class_labels[0] * 0 + 1).astype(jnp.float32)
    tpair = table[:_NUM_CLASSES].reshape(_NUM_CLASSES // 2, 2 * _HIDDEN) * scale
    return _embed(labels3, drops3, tpair, dr).T


# speedup vs baseline: 1.0180x; 1.0180x over previous
import jax
import jax.numpy as jnp
from jax import lax
from jax.experimental import pallas as pl
from jax.experimental.pallas import tpu as pltpu
from jax.experimental.pallas import tpu_sc as plsc

_NUM_CLASSES = 1000000
_HIDDEN = 64
_BATCH = 16384

_NC = 2
_NS = 16
_LANES = 16
_NW = _NC * _NS
_BPW = _BATCH // _NW
_CHUNK = 128
_NCHUNK = _BPW // _CHUNK


def _emb_kernel(labels_hbm, drops_hbm, table_hbm, out_hbm,
                drops_v, idx_v, dr_v, rows_v, gsem, dsem):
    wid = lax.axis_index("s") * _NC + lax.axis_index("c")
    base = wid * _BPW

    pltpu.sync_copy(drops_hbm.at[pl.ds(base, _BPW)], drops_v)

    pltpu.sync_copy(labels_hbm.at[wid], idx_v)

    pltpu.async_copy(table_hbm.at[pl.ds(_NUM_CLASSES, 1)], dr_v, dsem).wait()

    copies = [
        pltpu.async_copy(table_hbm.at[idx_v.at[j]], rows_v.at[j], gsem)
        for j in range(_NCHUNK)
    ]
    for cp in copies:
        cp.wait()

    drj = [dr_v[0, pl.ds(j * _LANES, _LANES)] for j in range(_HIDDEN // _LANES)]

    def fix(g, carry):
        dvec = drops_v[pl.ds(g * _LANES, _LANES)]
        for k in range(_LANES):
            @pl.when(dvec[k] == 1)
            def _():
                i = g * _LANES + k
                row = rows_v.at[i // _CHUNK].at[lax.rem(i, _CHUNK)]
                for j in range(_HIDDEN // _LANES):
                    row[pl.ds(j * _LANES, _LANES)] = drj[j]
        return carry

    lax.fori_loop(0, _BPW // _LANES, fix, 0)

    for j in range(_NCHUNK):
        pltpu.sync_copy(rows_v.at[j],
                        out_hbm.at[pl.ds(base + j * _CHUNK, _CHUNK)])


@jax.jit
def _embed(labels, drops, table):
    mesh = plsc.VectorSubcoreMesh(core_axis_name="c", subcore_axis_name="s")
    return pl.kernel(
        _emb_kernel,
        mesh=mesh,
        out_type=jax.ShapeDtypeStruct((_BATCH, _HIDDEN), jnp.float32),
        scratch_types=[
            pltpu.VMEM((_BPW,), jnp.int32),
            pltpu.VMEM((_NCHUNK, _CHUNK), jnp.int32),
            pltpu.VMEM((1, _HIDDEN), jnp.float32),
            pltpu.VMEM((_NCHUNK, _CHUNK, _HIDDEN), jnp.float32),
            pltpu.SemaphoreType.DMA,
            pltpu.SemaphoreType.DMA,
        ],
        compiler_params=pltpu.CompilerParams(use_tc_tiling_on_sc=False),
    )(labels, drops, table)


def kernel(class_labels, train, force_drop_ids, table):
    del train
    return _embed(class_labels.astype(jnp.int32).reshape(_NW, _NCHUNK, _CHUNK),
                  force_drop_ids.astype(jnp.int32), table)
